# Initial kernel scaffold; baseline (speedup 1.0000x reference)
import functools

import jax
import jax.numpy as jnp
from jax import lax
from jax.experimental import pallas as pl
from jax.experimental.pallas import tpu as pltpu
from jax.experimental.pallas import tpu_sc as plsc

N = 10000
E = 320000
D = 128

NC = 2
NS = 16
NW = NC * NS
EPW = E // NW
K = 80
NCHUNK = EPW // K
RPW = N // NS
ZR = 125
DEGW = 16

_mesh = plsc.VectorSubcoreMesh(core_axis_name="c", subcore_axis_name="s")


def _wid():
    return lax.axis_index("s") * NC + lax.axis_index("c")


@functools.partial(
    pl.kernel,
    out_type=jax.ShapeDtypeStruct((NC, N, DEGW), jnp.float32),
    mesh=_mesh,
    scratch_types=[
        pltpu.VMEM((NCHUNK, K), jnp.int32),
        pltpu.VMEM((K, DEGW), jnp.float32),
        pltpu.VMEM((RPW, DEGW), jnp.float32),
        pltpu.VMEM_SHARED((N, DEGW), jnp.float32),
    ],
)
def _deg(dst2_hbm, dp_hbm, idx_v, ones_v, zero_v, acc):
    c = lax.axis_index("c")
    s = lax.axis_index("s")
    wid = _wid()
    one16 = jnp.ones((DEGW,), jnp.float32)
    zero16 = jnp.zeros((DEGW,), jnp.float32)

    @pl.loop(0, K)
    def _(i):
        ones_v[i, :] = one16

    @pl.loop(0, RPW)
    def _(i):
        zero_v[i, :] = zero16

    pltpu.sync_copy(zero_v, acc.at[pl.ds(s * RPW, RPW)])
    pltpu.sync_copy(dst2_hbm.at[pl.ds(wid * NCHUNK, NCHUNK)], idx_v)
    plsc.subcore_barrier()

    @pl.loop(0, NCHUNK)
    def _(j):
        pltpu.sync_copy(ones_v, acc.at[idx_v.at[j]], add=True)

    plsc.subcore_barrier()
    pltpu.sync_copy(acc.at[pl.ds(s * RPW, RPW)], dp_hbm.at[c, pl.ds(s * RPW, RPW)])


@functools.partial(
    pl.kernel,
    out_type=jax.ShapeDtypeStruct((NC, N, D), jnp.float32),
    mesh=_mesh,
    scratch_types=[
        pltpu.VMEM((NCHUNK, K), jnp.int32),
        pltpu.VMEM((NCHUNK, K), jnp.int32),
        pltpu.VMEM((K, D), jnp.float32),
        pltpu.VMEM((ZR, D), jnp.float32),
        pltpu.VMEM_SHARED((N, D), jnp.float32),
    ],
)
def _conv(g_hbm, src2_hbm, dst2_hbm, p_hbm, sidx_v, didx_v, rows_v, zrow_v, acc):
    c = lax.axis_index("c")
    s = lax.axis_index("s")
    wid = _wid()
    zero16 = jnp.zeros((16,), jnp.float32)

    @pl.loop(0, ZR)
    def _(i):
        @pl.loop(0, D // 16)
        def _(k):
            zrow_v[i, pl.ds(k * 16, 16)] = zero16

    @pl.loop(0, RPW // ZR)
    def _(t):
        pltpu.sync_copy(zrow_v, acc.at[pl.ds(s * RPW + t * ZR, ZR)])

    pltpu.sync_copy(src2_hbm.at[pl.ds(wid * NCHUNK, NCHUNK)], sidx_v)
    pltpu.sync_copy(dst2_hbm.at[pl.ds(wid * NCHUNK, NCHUNK)], didx_v)
    plsc.subcore_barrier()

    @pl.loop(0, NCHUNK)
    def _(j):
        pltpu.sync_copy(g_hbm.at[sidx_v.at[j]], rows_v)
        pltpu.sync_copy(rows_v, acc.at[didx_v.at[j]], add=True)

    plsc.subcore_barrier()
    pltpu.sync_copy(acc.at[pl.ds(s * RPW, RPW)], p_hbm.at[c, pl.ds(s * RPW, RPW)])


@functools.partial(
    pl.kernel,
    out_type=jax.ShapeDtypeStruct((E,), jnp.float32),
    mesh=_mesh,
    scratch_types=[
        pltpu.VMEM((N,), jnp.float32),
        pltpu.VMEM((N,), jnp.float32),
        pltpu.VMEM((EPW,), jnp.int32),
        pltpu.VMEM((EPW,), jnp.int32),
        pltpu.VMEM((EPW,), jnp.float32),
    ],
)
def _escore(sa_hbm, sb_hbm, src_hbm, dst_hbm, out_hbm, sa_v, sb_v, src_v, dst_v, out_v):
    wid = _wid()
    base = wid * EPW
    pltpu.sync_copy(sa_hbm, sa_v)
    pltpu.sync_copy(sb_hbm, sb_v)
    pltpu.sync_copy(src_hbm.at[pl.ds(base, EPW)], src_v)
    pltpu.sync_copy(dst_hbm.at[pl.ds(base, EPW)], dst_v)

    @pl.loop(0, EPW, step=16)
    def _(j):
        iv_s = src_v[pl.ds(j, 16)]
        iv_d = dst_v[pl.ds(j, 16)]
        va = plsc.load_gather(sa_v, [iv_s])
        vb = plsc.load_gather(sb_v, [iv_d])
        out_v[pl.ds(j, 16)] = va + vb

    pltpu.sync_copy(out_v, out_hbm.at[pl.ds(base, EPW)])


_BLK = 1000
_NBLK = N // _BLK


def _mm_body(x_ref, w_ref, o_ref):
    o_ref[...] = jnp.dot(x_ref[...], w_ref[...],
                         preferred_element_type=jnp.float32,
                         precision=lax.Precision.HIGHEST)


_mm = pl.pallas_call(
    _mm_body,
    grid=(_NBLK,),
    in_specs=[pl.BlockSpec((_BLK, D), lambda i: (i, 0)),
              pl.BlockSpec((D, D), lambda i: (0, 0))],
    out_specs=pl.BlockSpec((_BLK, D), lambda i: (i, 0)),
    out_shape=jax.ShapeDtypeStruct((N, D), jnp.float32),
)


def _s1_body(dp0_ref, dp1_ref, h_ref, g_ref, dinv_ref):
    deg = dp0_ref[:, 0:1] + dp1_ref[:, 0:1] + 1.0
    dinv = lax.rsqrt(jnp.maximum(deg, 1e-12))
    dinv_ref[...] = dinv
    g_ref[...] = h_ref[...] * dinv


_s1 = pl.pallas_call(
    _s1_body,
    grid=(_NBLK,),
    in_specs=[pl.BlockSpec((_BLK, DEGW), lambda i: (i, 0)),
              pl.BlockSpec((_BLK, DEGW), lambda i: (i, 0)),
              pl.BlockSpec((_BLK, D), lambda i: (i, 0))],
    out_specs=[pl.BlockSpec((_BLK, D), lambda i: (i, 0)),
               pl.BlockSpec((_BLK, 1), lambda i: (i, 0))],
    out_shape=[jax.ShapeDtypeStruct((N, D), jnp.float32),
               jax.ShapeDtypeStruct((N, 1), jnp.float32)],
)


def _mid_body(p0_ref, p1_ref, g_ref, dinv_ref, b_ref, w_ref, gout_ref):
    h = jnp.maximum(
        dinv_ref[...] * (p0_ref[...] + p1_ref[...] + g_ref[...]) + b_ref[...], 0.0)
    gout_ref[...] = dinv_ref[...] * jnp.dot(
        h, w_ref[...], preferred_element_type=jnp.float32,
        precision=lax.Precision.HIGHEST)


_mid = pl.pallas_call(
    _mid_body,
    grid=(_NBLK,),
    in_specs=[pl.BlockSpec((_BLK, D), lambda i: (i, 0)),
              pl.BlockSpec((_BLK, D), lambda i: (i, 0)),
              pl.BlockSpec((_BLK, D), lambda i: (i, 0)),
              pl.BlockSpec((_BLK, 1), lambda i: (i, 0)),
              pl.BlockSpec((1, D), lambda i: (0, 0)),
              pl.BlockSpec((D, D), lambda i: (0, 0))],
    out_specs=pl.BlockSpec((_BLK, D), lambda i: (i, 0)),
    out_shape=jax.ShapeDtypeStruct((N, D), jnp.float32),
)


def _fin_body(p0_ref, p1_ref, g_ref, dinv_ref, b_ref, wab_ref, blin_ref,
              sa_ref, sb_ref):
    h = jnp.maximum(
        dinv_ref[...] * (p0_ref[...] + p1_ref[...] + g_ref[...]) + b_ref[...], 0.0)
    s = jnp.dot(h, wab_ref[...], preferred_element_type=jnp.float32,
                precision=lax.Precision.HIGHEST)
    sa_ref[...] = s[:, 0:1] + blin_ref[0, 0]
    sb_ref[...] = s[:, 1:2]


_fin = pl.pallas_call(
    _fin_body,
    grid=(_NBLK,),
    in_specs=[pl.BlockSpec((_BLK, D), lambda i: (i, 0)),
              pl.BlockSpec((_BLK, D), lambda i: (i, 0)),
              pl.BlockSpec((_BLK, D), lambda i: (i, 0)),
              pl.BlockSpec((_BLK, 1), lambda i: (i, 0)),
              pl.BlockSpec((1, D), lambda i: (0, 0)),
              pl.BlockSpec((D, 2), lambda i: (0, 0)),
              pl.BlockSpec((1, 1), lambda i: (0, 0))],
    out_specs=[pl.BlockSpec((_BLK, 1), lambda i: (i, 0)),
               pl.BlockSpec((_BLK, 1), lambda i: (i, 0))],
    out_shape=[jax.ShapeDtypeStruct((N, 1), jnp.float32),
               jax.ShapeDtypeStruct((N, 1), jnp.float32)],
)


def kernel(x, edge_index, W1, b1, W2, b2, Wlin, blin):
    src = edge_index[0]
    dst = edge_index[1]
    src2 = src.reshape(E // K, K)
    dst2 = dst.reshape(E // K, K)

    dp = _deg(dst2)
    h1raw = _mm(x, W1)
    g1, dinv = _s1(dp[0], dp[1], h1raw)

    p = _conv(g1, src2, dst2)
    g2 = _mid(p[0], p[1], g1, dinv, b1.reshape(1, D), W2)

    p2 = _conv(g2, src2, dst2)
    wab = jnp.concatenate([Wlin[:D], Wlin[D:]], axis=1)
    sa, sb = _fin(p2[0], p2[1], g2, dinv, b2.reshape(1, D), wab,
                  blin.reshape(1, 1))

    return _escore(sa.reshape(N), sb.reshape(N), src, dst)


# speedup vs baseline: 20.7441x; 20.7441x over previous
import dataclasses
import functools

import jax
import jax.numpy as jnp
from jax import lax
from jax.experimental import pallas as pl
from jax.experimental.pallas import tpu as pltpu
from jax.experimental.pallas import tpu_sc as plsc

N = 10000
NP = 10240
E = 320000
D = 128

NC = 2
NS = 16
NW = NC * NS
EPW = E // NW
K = 80
NCHUNK = EPW // K
RPW = NP // NS
DEGW = 16

_mesh = plsc.VectorSubcoreMesh(core_axis_name="c", subcore_axis_name="s",
                               num_cores=NC, num_subcores=NS)

_cp = pltpu.CompilerParams()
if "needs_layout_passes" in pltpu.CompilerParams.__dataclass_fields__:
    _cp = dataclasses.replace(_cp, needs_layout_passes=False)


def _wid():
    return lax.axis_index("s") * NC + lax.axis_index("c")


@functools.partial(
    pl.kernel,
    out_type=jax.ShapeDtypeStruct((NW, NP), jnp.float32),
    mesh=_mesh,
    scratch_types=[
        pltpu.VMEM((EPW,), jnp.int32),
        pltpu.VMEM((NP,), jnp.float32),
    ],
    compiler_params=_cp,
)
def _deg(dst_hbm, dp_hbm, idx_v, deg_v):
    wid = _wid()
    zero16 = jnp.zeros((16,), jnp.float32)
    one16 = jnp.ones((16,), jnp.float32)

    @pl.loop(0, NP, step=16)
    def _(i):
        deg_v[pl.ds(i, 16)] = zero16

    pltpu.sync_copy(dst_hbm.at[pl.ds(wid * EPW, EPW)], idx_v)

    @pl.loop(0, EPW, step=16)
    def _(j):
        iv = idx_v[pl.ds(j, 16)]
        plsc.addupdate_scatter(deg_v, [iv], one16)

    pltpu.sync_copy(deg_v, dp_hbm.at[wid])


@functools.partial(
    pl.kernel,
    out_type=jax.ShapeDtypeStruct((NC, NP, D), jnp.float32),
    mesh=_mesh,
    scratch_types=[
        pltpu.VMEM((NCHUNK, K), jnp.int32),
        pltpu.VMEM((NCHUNK, K), jnp.int32),
        pltpu.VMEM((K, D), jnp.float32),
        pltpu.VMEM_SHARED((NP, D), jnp.float32),
    ],
)
def _conv(g_hbm, zeros_hbm, src2_hbm, dst2_hbm, p_hbm, sidx_v, didx_v,
          rows_v, acc):
    c = lax.axis_index("c")
    s = lax.axis_index("s")
    wid = _wid()

    @pl.when(c == 0)
    def _():
        pltpu.sync_copy(g_hbm.at[pl.ds(s * RPW, RPW)],
                        acc.at[pl.ds(s * RPW, RPW)])

    @pl.when(c == 1)
    def _():
        pltpu.sync_copy(zeros_hbm.at[pl.ds(s * RPW, RPW)],
                        acc.at[pl.ds(s * RPW, RPW)])

    pltpu.sync_copy(src2_hbm.at[wid], sidx_v)
    pltpu.sync_copy(dst2_hbm.at[wid], didx_v)
    plsc.subcore_barrier()

    @pl.loop(0, NCHUNK)
    def _(j):
        pltpu.sync_copy(g_hbm.at[sidx_v.at[j]], rows_v)
        pltpu.sync_copy(rows_v, acc.at[didx_v.at[j]], add=True)

    plsc.subcore_barrier()
    pltpu.sync_copy(acc.at[pl.ds(s * RPW, RPW)],
                    p_hbm.at[c, pl.ds(s * RPW, RPW)])


@functools.partial(
    pl.kernel,
    out_type=jax.ShapeDtypeStruct((E,), jnp.float32),
    mesh=_mesh,
    scratch_types=[
        pltpu.VMEM((NP,), jnp.float32),
        pltpu.VMEM((NP,), jnp.float32),
        pltpu.VMEM((EPW,), jnp.int32),
        pltpu.VMEM((EPW,), jnp.int32),
        pltpu.VMEM((EPW,), jnp.float32),
    ],
    compiler_params=_cp,
)
def _escore(sa_hbm, sb_hbm, src_hbm, dst_hbm, out_hbm, sa_v, sb_v, src_v,
            dst_v, out_v):
    wid = _wid()
    base = wid * EPW
    pltpu.sync_copy(sa_hbm, sa_v)
    pltpu.sync_copy(sb_hbm, sb_v)
    pltpu.sync_copy(src_hbm.at[pl.ds(base, EPW)], src_v)
    pltpu.sync_copy(dst_hbm.at[pl.ds(base, EPW)], dst_v)

    @pl.loop(0, EPW, step=16)
    def _(j):
        iv_s = src_v[pl.ds(j, 16)]
        iv_d = dst_v[pl.ds(j, 16)]
        va = plsc.load_gather(sa_v, [iv_s])
        vb = plsc.load_gather(sb_v, [iv_d])
        out_v[pl.ds(j, 16)] = va + vb

    pltpu.sync_copy(out_v, out_hbm.at[pl.ds(base, EPW)])


_BLK = 1024
_NBLK = NP // _BLK


def _mm_body(x_ref, w_ref, o_ref):
    o_ref[...] = jnp.dot(x_ref[...], w_ref[...],
                         preferred_element_type=jnp.float32)


_mm = pl.pallas_call(
    _mm_body,
    grid=(_NBLK,),
    in_specs=[pl.BlockSpec((_BLK, D), lambda i: (i, 0)),
              pl.BlockSpec((D, D), lambda i: (0, 0))],
    out_specs=pl.BlockSpec((_BLK, D), lambda i: (i, 0)),
    out_shape=jax.ShapeDtypeStruct((NP, D), jnp.float32),
)


def _degsum_body(dp_ref, o_ref):
    o_ref[...] = jnp.sum(dp_ref[...], axis=0, keepdims=True) + 1.0


_degsum = pl.pallas_call(
    _degsum_body,
    grid=(_NBLK,),
    in_specs=[pl.BlockSpec((NW, _BLK), lambda i: (0, i))],
    out_specs=pl.BlockSpec((1, _BLK), lambda i: (0, i)),
    out_shape=jax.ShapeDtypeStruct((1, NP), jnp.float32),
)


def _s1_body(degcol_ref, h_ref, g_ref, dinv_ref):
    dinv = lax.rsqrt(jnp.maximum(degcol_ref[...], 1e-12))
    dinv_ref[...] = dinv
    g_ref[...] = h_ref[...] * dinv


_s1 = pl.pallas_call(
    _s1_body,
    grid=(_NBLK,),
    in_specs=[pl.BlockSpec((_BLK, 1), lambda i: (i, 0)),
              pl.BlockSpec((_BLK, D), lambda i: (i, 0))],
    out_specs=[pl.BlockSpec((_BLK, D), lambda i: (i, 0)),
               pl.BlockSpec((_BLK, 1), lambda i: (i, 0))],
    out_shape=[jax.ShapeDtypeStruct((NP, D), jnp.float32),
               jax.ShapeDtypeStruct((NP, 1), jnp.float32)],
)


def _mid_body(p0_ref, p1_ref, dinv_ref, b_ref, w_ref, gout_ref):
    h = jnp.maximum(
        dinv_ref[...] * (p0_ref[...] + p1_ref[...]) + b_ref[...], 0.0)
    gout_ref[...] = dinv_ref[...] * jnp.dot(
        h, w_ref[...], preferred_element_type=jnp.float32)


_mid = pl.pallas_call(
    _mid_body,
    grid=(_NBLK,),
    in_specs=[pl.BlockSpec((_BLK, D), lambda i: (i, 0)),
              pl.BlockSpec((_BLK, D), lambda i: (i, 0)),
              pl.BlockSpec((_BLK, 1), lambda i: (i, 0)),
              pl.BlockSpec((1, D), lambda i: (0, 0)),
              pl.BlockSpec((D, D), lambda i: (0, 0))],
    out_specs=pl.BlockSpec((_BLK, D), lambda i: (i, 0)),
    out_shape=jax.ShapeDtypeStruct((NP, D), jnp.float32),
)


def _fin_body(p0_ref, p1_ref, dinv_ref, b_ref, wab_ref, blin_ref,
              sa_ref, sb_ref):
    h = jnp.maximum(
        dinv_ref[...] * (p0_ref[...] + p1_ref[...]) + b_ref[...], 0.0)
    s = jnp.dot(h, wab_ref[...], preferred_element_type=jnp.float32)
    sa_ref[...] = s[:, 0:1] + blin_ref[0, 0]
    sb_ref[...] = s[:, 1:2]


_fin = pl.pallas_call(
    _fin_body,
    grid=(_NBLK,),
    in_specs=[pl.BlockSpec((_BLK, D), lambda i: (i, 0)),
              pl.BlockSpec((_BLK, D), lambda i: (i, 0)),
              pl.BlockSpec((_BLK, 1), lambda i: (i, 0)),
              pl.BlockSpec((1, D), lambda i: (0, 0)),
              pl.BlockSpec((D, 2), lambda i: (0, 0)),
              pl.BlockSpec((1, 1), lambda i: (0, 0))],
    out_specs=[pl.BlockSpec((_BLK, 1), lambda i: (i, 0)),
               pl.BlockSpec((_BLK, 1), lambda i: (i, 0))],
    out_shape=[jax.ShapeDtypeStruct((NP, 1), jnp.float32),
               jax.ShapeDtypeStruct((NP, 1), jnp.float32)],
)


def kernel(x, edge_index, W1, b1, W2, b2, Wlin, blin):
    src = edge_index[0]
    dst = edge_index[1]
    src2 = src.reshape(NW, NCHUNK, K)
    dst2 = dst.reshape(NW, NCHUNK, K)
    xp = jnp.pad(x, ((0, NP - N), (0, 0)))
    zeros_d = jnp.zeros((NP, D), jnp.float32)

    dp = _deg(dst)
    h1raw = _mm(xp, W1)
    degcol = _degsum(dp).reshape(NP, 1)
    g1, dinv = _s1(degcol, h1raw)

    p = _conv(g1, zeros_d, src2, dst2)
    g2 = _mid(p[0], p[1], dinv, b1.reshape(1, D), W2)

    p2 = _conv(g2, zeros_d, src2, dst2)
    wab = jnp.concatenate([Wlin[:D], Wlin[D:]], axis=1)
    sa, sb = _fin(p2[0], p2[1], dinv, b2.reshape(1, D), wab,
                  blin.reshape(1, 1))

    return _escore(sa.reshape(NP), sb.reshape(NP), src, dst)


# speedup vs baseline: 24.6582x; 1.1887x over previous
import dataclasses
import functools

import jax
import jax.numpy as jnp
from jax import lax
from jax.experimental import pallas as pl
from jax.experimental.pallas import tpu as pltpu
from jax.experimental.pallas import tpu_sc as plsc

N = 10000
NP = 10240
E = 320000
D = 128

NC = 2
NS = 16
NW = NC * NS
EPW = E // NW
K = 80
NCHUNK = EPW // K
NPH = 5
CPP = NCHUNK // NPH
RPW = NP // NS
DEGW = 16

_mesh = plsc.VectorSubcoreMesh(core_axis_name="c", subcore_axis_name="s",
                               num_cores=NC, num_subcores=NS)

_cp = pltpu.CompilerParams()
if "needs_layout_passes" in pltpu.CompilerParams.__dataclass_fields__:
    _cp = dataclasses.replace(_cp, needs_layout_passes=False)


def _wid():
    return lax.axis_index("s") * NC + lax.axis_index("c")


@functools.partial(
    pl.kernel,
    out_type=jax.ShapeDtypeStruct((NW, NP), jnp.float32),
    mesh=_mesh,
    scratch_types=[
        pltpu.VMEM((EPW,), jnp.int32),
        pltpu.VMEM((NP,), jnp.float32),
    ],
    compiler_params=_cp,
)
def _deg(dst_hbm, dp_hbm, idx_v, deg_v):
    wid = _wid()
    zero16 = jnp.zeros((16,), jnp.float32)
    one16 = jnp.ones((16,), jnp.float32)

    @pl.loop(0, NP, step=16)
    def _(i):
        deg_v[pl.ds(i, 16)] = zero16

    pltpu.sync_copy(dst_hbm.at[pl.ds(wid * EPW, EPW)], idx_v)

    @pl.loop(0, EPW, step=16)
    def _(j):
        iv = idx_v[pl.ds(j, 16)]
        plsc.addupdate_scatter(deg_v, [iv], one16)

    pltpu.sync_copy(deg_v, dp_hbm.at[wid])


@functools.partial(
    pl.kernel,
    out_type=jax.ShapeDtypeStruct((NC, NP, D), jnp.float32),
    mesh=_mesh,
    scratch_types=[
        pltpu.VMEM((CPP, K), jnp.int32),
        pltpu.VMEM((CPP, K), jnp.int32),
        pltpu.VMEM((K, D), jnp.float32),
        pltpu.VMEM((K, D), jnp.float32),
        pltpu.SemaphoreType.DMA,
        pltpu.SemaphoreType.DMA,
        pltpu.VMEM_SHARED((NP, D), jnp.float32),
    ],
)
def _conv(g_hbm, zeros_hbm, src4_hbm, dst4_hbm, p_hbm, sidx_v, didx_v,
          r0, r1, gsem0, gsem1, acc):
    c = lax.axis_index("c")
    s = lax.axis_index("s")
    wid = _wid()

    @pl.when(c == 0)
    def _():
        pltpu.sync_copy(g_hbm.at[pl.ds(s * RPW, RPW)],
                        acc.at[pl.ds(s * RPW, RPW)])

    @pl.when(c == 1)
    def _():
        pltpu.sync_copy(zeros_hbm.at[pl.ds(s * RPW, RPW)],
                        acc.at[pl.ds(s * RPW, RPW)])

    plsc.subcore_barrier()

    @pl.loop(0, NPH)
    def _(t):
        pltpu.sync_copy(src4_hbm.at[wid, t], sidx_v)
        pltpu.sync_copy(dst4_hbm.at[wid, t], didx_v)
        pltpu.async_copy(g_hbm.at[sidx_v.at[0]], r0, gsem0)

        @pl.loop(0, (CPP - 1) // 2)
        def _(jj):
            j0 = 2 * jj
            pltpu.make_async_copy(g_hbm.at[sidx_v.at[j0]], r0, gsem0).wait()
            pltpu.async_copy(g_hbm.at[sidx_v.at[j0 + 1]], r1, gsem1)
            pltpu.sync_copy(r0, acc.at[didx_v.at[j0]], add=True)
            pltpu.make_async_copy(g_hbm.at[sidx_v.at[j0 + 1]], r1, gsem1).wait()
            pltpu.async_copy(g_hbm.at[sidx_v.at[j0 + 2]], r0, gsem0)
            pltpu.sync_copy(r1, acc.at[didx_v.at[j0 + 1]], add=True)

        pltpu.make_async_copy(g_hbm.at[sidx_v.at[CPP - 1]], r0, gsem0).wait()
        pltpu.sync_copy(r0, acc.at[didx_v.at[CPP - 1]], add=True)

    plsc.subcore_barrier()
    pltpu.sync_copy(acc.at[pl.ds(s * RPW, RPW)],
                    p_hbm.at[c, pl.ds(s * RPW, RPW)])


@functools.partial(
    pl.kernel,
    out_type=jax.ShapeDtypeStruct((E,), jnp.float32),
    mesh=_mesh,
    scratch_types=[
        pltpu.VMEM((NP,), jnp.float32),
        pltpu.VMEM((NP,), jnp.float32),
        pltpu.VMEM((EPW,), jnp.int32),
        pltpu.VMEM((EPW,), jnp.int32),
        pltpu.VMEM((EPW,), jnp.float32),
    ],
    compiler_params=_cp,
)
def _escore(sa_hbm, sb_hbm, src_hbm, dst_hbm, out_hbm, sa_v, sb_v, src_v,
            dst_v, out_v):
    wid = _wid()
    base = wid * EPW
    pltpu.sync_copy(sa_hbm, sa_v)
    pltpu.sync_copy(sb_hbm, sb_v)
    pltpu.sync_copy(src_hbm.at[pl.ds(base, EPW)], src_v)
    pltpu.sync_copy(dst_hbm.at[pl.ds(base, EPW)], dst_v)

    @pl.loop(0, EPW, step=16)
    def _(j):
        iv_s = src_v[pl.ds(j, 16)]
        iv_d = dst_v[pl.ds(j, 16)]
        va = plsc.load_gather(sa_v, [iv_s])
        vb = plsc.load_gather(sb_v, [iv_d])
        out_v[pl.ds(j, 16)] = va + vb

    pltpu.sync_copy(out_v, out_hbm.at[pl.ds(base, EPW)])


_BLK = 1024
_NBLK = NP // _BLK


def _mm_body(x_ref, w_ref, o_ref):
    o_ref[...] = jnp.dot(x_ref[...], w_ref[...],
                         preferred_element_type=jnp.float32)


_mm = pl.pallas_call(
    _mm_body,
    grid=(_NBLK,),
    in_specs=[pl.BlockSpec((_BLK, D), lambda i: (i, 0)),
              pl.BlockSpec((D, D), lambda i: (0, 0))],
    out_specs=pl.BlockSpec((_BLK, D), lambda i: (i, 0)),
    out_shape=jax.ShapeDtypeStruct((NP, D), jnp.float32),
)


def _degsum_body(dp_ref, o_ref):
    o_ref[...] = jnp.sum(dp_ref[...], axis=0, keepdims=True) + 1.0


_degsum = pl.pallas_call(
    _degsum_body,
    grid=(_NBLK,),
    in_specs=[pl.BlockSpec((NW, _BLK), lambda i: (0, i))],
    out_specs=pl.BlockSpec((1, _BLK), lambda i: (0, i)),
    out_shape=jax.ShapeDtypeStruct((1, NP), jnp.float32),
)


def _s1_body(degcol_ref, h_ref, g_ref, dinv_ref):
    dinv = lax.rsqrt(jnp.maximum(degcol_ref[...], 1e-12))
    dinv_ref[...] = dinv
    g_ref[...] = h_ref[...] * dinv


_s1 = pl.pallas_call(
    _s1_body,
    grid=(_NBLK,),
    in_specs=[pl.BlockSpec((_BLK, 1), lambda i: (i, 0)),
              pl.BlockSpec((_BLK, D), lambda i: (i, 0))],
    out_specs=[pl.BlockSpec((_BLK, D), lambda i: (i, 0)),
               pl.BlockSpec((_BLK, 1), lambda i: (i, 0))],
    out_shape=[jax.ShapeDtypeStruct((NP, D), jnp.float32),
               jax.ShapeDtypeStruct((NP, 1), jnp.float32)],
)


def _mid_body(p0_ref, p1_ref, dinv_ref, b_ref, w_ref, gout_ref):
    h = jnp.maximum(
        dinv_ref[...] * (p0_ref[...] + p1_ref[...]) + b_ref[...], 0.0)
    gout_ref[...] = dinv_ref[...] * jnp.dot(
        h, w_ref[...], preferred_element_type=jnp.float32)


_mid = pl.pallas_call(
    _mid_body,
    grid=(_NBLK,),
    in_specs=[pl.BlockSpec((_BLK, D), lambda i: (i, 0)),
              pl.BlockSpec((_BLK, D), lambda i: (i, 0)),
              pl.BlockSpec((_BLK, 1), lambda i: (i, 0)),
              pl.BlockSpec((1, D), lambda i: (0, 0)),
              pl.BlockSpec((D, D), lambda i: (0, 0))],
    out_specs=pl.BlockSpec((_BLK, D), lambda i: (i, 0)),
    out_shape=jax.ShapeDtypeStruct((NP, D), jnp.float32),
)


def _fin_body(p0_ref, p1_ref, dinv_ref, b_ref, wab_ref, blin_ref,
              sa_ref, sb_ref):
    h = jnp.maximum(
        dinv_ref[...] * (p0_ref[...] + p1_ref[...]) + b_ref[...], 0.0)
    s = jnp.dot(h, wab_ref[...], preferred_element_type=jnp.float32)
    sa_ref[...] = s[:, 0:1] + blin_ref[0, 0]
    sb_ref[...] = s[:, 1:2]


_fin = pl.pallas_call(
    _fin_body,
    grid=(_NBLK,),
    in_specs=[pl.BlockSpec((_BLK, D), lambda i: (i, 0)),
              pl.BlockSpec((_BLK, D), lambda i: (i, 0)),
              pl.BlockSpec((_BLK, 1), lambda i: (i, 0)),
              pl.BlockSpec((1, D), lambda i: (0, 0)),
              pl.BlockSpec((D, 2), lambda i: (0, 0)),
              pl.BlockSpec((1, 1), lambda i: (0, 0))],
    out_specs=[pl.BlockSpec((_BLK, 1), lambda i: (i, 0)),
               pl.BlockSpec((_BLK, 1), lambda i: (i, 0))],
    out_shape=[jax.ShapeDtypeStruct((NP, 1), jnp.float32),
               jax.ShapeDtypeStruct((NP, 1), jnp.float32)],
)


def kernel(x, edge_index, W1, b1, W2, b2, Wlin, blin):
    src = edge_index[0]
    dst = edge_index[1]
    src4 = src.reshape(NW, NPH, CPP, K)
    dst4 = dst.reshape(NW, NPH, CPP, K)
    xp = jnp.pad(x, ((0, NP - N), (0, 0)))
    zeros_d = jnp.zeros((NP, D), jnp.float32)

    dp = _deg(dst)
    h1raw = _mm(xp, W1)
    degcol = _degsum(dp).reshape(NP, 1)
    g1, dinv = _s1(degcol, h1raw)

    p = _conv(g1, zeros_d, src4, dst4)
    g2 = _mid(p[0], p[1], dinv, b1.reshape(1, D), W2)

    p2 = _conv(g2, zeros_d, src4, dst4)
    wab = jnp.concatenate([Wlin[:D], Wlin[D:]], axis=1)
    sa, sb = _fin(p2[0], p2[1], dinv, b2.reshape(1, D), wab,
                  blin.reshape(1, 1))

    return _escore(sa.reshape(NP), sb.reshape(NP), src, dst)


# speedup vs baseline: 32.6460x; 1.3239x over previous
import dataclasses
import functools

import jax
import jax.numpy as jnp
from jax import lax
from jax.experimental import pallas as pl
from jax.experimental.pallas import tpu as pltpu
from jax.experimental.pallas import tpu_sc as plsc

N = 10000
NP = 10240
E = 320000
D = 128

NC = 2
NS = 16
NW = NC * NS
EPW = E // NW
K = 40
NCHUNK = EPW // K
NPH = 5
CPP = NCHUNK // NPH
NBUF = 5
RPW = NP // NS
DEGW = 16

_mesh = plsc.VectorSubcoreMesh(core_axis_name="c", subcore_axis_name="s",
                               num_cores=NC, num_subcores=NS)

_cp = pltpu.CompilerParams()
if "needs_layout_passes" in pltpu.CompilerParams.__dataclass_fields__:
    _cp = dataclasses.replace(_cp, needs_layout_passes=False)


def _wid():
    return lax.axis_index("s") * NC + lax.axis_index("c")


@functools.partial(
    pl.kernel,
    out_type=jax.ShapeDtypeStruct((NW, NP), jnp.float32),
    mesh=_mesh,
    scratch_types=[
        pltpu.VMEM((EPW,), jnp.int32),
        pltpu.VMEM((NP,), jnp.float32),
    ],
    compiler_params=_cp,
)
def _deg(dst_hbm, dp_hbm, idx_v, deg_v):
    wid = _wid()
    zero16 = jnp.zeros((16,), jnp.float32)
    one16 = jnp.ones((16,), jnp.float32)

    @pl.loop(0, NP, step=16)
    def _(i):
        deg_v[pl.ds(i, 16)] = zero16

    pltpu.sync_copy(dst_hbm.at[pl.ds(wid * EPW, EPW)], idx_v)

    @pl.loop(0, EPW, step=16)
    def _(j):
        iv = idx_v[pl.ds(j, 16)]
        plsc.addupdate_scatter(deg_v, [iv], one16)

    pltpu.sync_copy(deg_v, dp_hbm.at[wid])


@functools.partial(
    pl.kernel,
    out_type=jax.ShapeDtypeStruct((NC, NP, D), jnp.float32),
    mesh=_mesh,
    scratch_types=[
        pltpu.VMEM((CPP, K), jnp.int32),
        pltpu.VMEM((CPP, K), jnp.int32),
    ] + [pltpu.VMEM((K, D), jnp.float32)] * NBUF
      + [pltpu.SemaphoreType.DMA] * NBUF
      + [pltpu.VMEM_SHARED((NP, D), jnp.float32)],
)
def _conv(g_hbm, zeros_hbm, src4_hbm, dst4_hbm, p_hbm, sidx_v, didx_v,
          *rest):
    bufs = rest[:NBUF]
    sems = rest[NBUF:2 * NBUF]
    acc = rest[2 * NBUF]
    c = lax.axis_index("c")
    s = lax.axis_index("s")
    wid = _wid()

    @pl.when(c == 0)
    def _():
        pltpu.sync_copy(g_hbm.at[pl.ds(s * RPW, RPW)],
                        acc.at[pl.ds(s * RPW, RPW)])

    @pl.when(c == 1)
    def _():
        pltpu.sync_copy(zeros_hbm.at[pl.ds(s * RPW, RPW)],
                        acc.at[pl.ds(s * RPW, RPW)])

    plsc.subcore_barrier()

    @pl.loop(0, NPH)
    def _(t):
        pltpu.sync_copy(src4_hbm.at[wid, t], sidx_v)
        pltpu.sync_copy(dst4_hbm.at[wid, t], didx_v)
        for b in range(NBUF - 1):
            pltpu.async_copy(g_hbm.at[sidx_v.at[b]], bufs[b], sems[b])

        @pl.loop(0, CPP // NBUF)
        def _(g):
            j0 = g * NBUF
            for b in range(NBUF):
                j = j0 + b
                pltpu.make_async_copy(
                    g_hbm.at[sidx_v.at[j]], bufs[b], sems[b]).wait()
                nxt = j + NBUF - 1

                @pl.when(nxt < CPP)
                def _(b=b, nxt=nxt):
                    pltpu.async_copy(
                        g_hbm.at[sidx_v.at[nxt]],
                        bufs[(b - 1) % NBUF], sems[(b - 1) % NBUF])

                pltpu.sync_copy(bufs[b], acc.at[didx_v.at[j]], add=True)

    plsc.subcore_barrier()
    pltpu.sync_copy(acc.at[pl.ds(s * RPW, RPW)],
                    p_hbm.at[c, pl.ds(s * RPW, RPW)])


@functools.partial(
    pl.kernel,
    out_type=jax.ShapeDtypeStruct((E,), jnp.float32),
    mesh=_mesh,
    scratch_types=[
        pltpu.VMEM((NP,), jnp.float32),
        pltpu.VMEM((NP,), jnp.float32),
        pltpu.VMEM((EPW,), jnp.int32),
        pltpu.VMEM((EPW,), jnp.int32),
        pltpu.VMEM((EPW,), jnp.float32),
    ],
    compiler_params=_cp,
)
def _escore(sa_hbm, sb_hbm, src_hbm, dst_hbm, out_hbm, sa_v, sb_v, src_v,
            dst_v, out_v):
    wid = _wid()
    base = wid * EPW
    pltpu.sync_copy(sa_hbm, sa_v)
    pltpu.sync_copy(sb_hbm, sb_v)
    pltpu.sync_copy(src_hbm.at[pl.ds(base, EPW)], src_v)
    pltpu.sync_copy(dst_hbm.at[pl.ds(base, EPW)], dst_v)

    @pl.loop(0, EPW, step=16)
    def _(j):
        iv_s = src_v[pl.ds(j, 16)]
        iv_d = dst_v[pl.ds(j, 16)]
        va = plsc.load_gather(sa_v, [iv_s])
        vb = plsc.load_gather(sb_v, [iv_d])
        out_v[pl.ds(j, 16)] = va + vb

    pltpu.sync_copy(out_v, out_hbm.at[pl.ds(base, EPW)])


_BLK = 1024
_NBLK = NP // _BLK


def _mm_body(x_ref, w_ref, o_ref):
    o_ref[...] = jnp.dot(x_ref[...], w_ref[...],
                         preferred_element_type=jnp.float32)


_mm = pl.pallas_call(
    _mm_body,
    grid=(_NBLK,),
    in_specs=[pl.BlockSpec((_BLK, D), lambda i: (i, 0)),
              pl.BlockSpec((D, D), lambda i: (0, 0))],
    out_specs=pl.BlockSpec((_BLK, D), lambda i: (i, 0)),
    out_shape=jax.ShapeDtypeStruct((NP, D), jnp.float32),
)


def _degsum_body(dp_ref, o_ref):
    o_ref[...] = jnp.sum(dp_ref[...], axis=0, keepdims=True) + 1.0


_degsum = pl.pallas_call(
    _degsum_body,
    grid=(_NBLK,),
    in_specs=[pl.BlockSpec((NW, _BLK), lambda i: (0, i))],
    out_specs=pl.BlockSpec((1, _BLK), lambda i: (0, i)),
    out_shape=jax.ShapeDtypeStruct((1, NP), jnp.float32),
)


def _s1_body(degcol_ref, h_ref, g_ref, dinv_ref):
    dinv = lax.rsqrt(jnp.maximum(degcol_ref[...], 1e-12))
    dinv_ref[...] = dinv
    g_ref[...] = h_ref[...] * dinv


_s1 = pl.pallas_call(
    _s1_body,
    grid=(_NBLK,),
    in_specs=[pl.BlockSpec((_BLK, 1), lambda i: (i, 0)),
              pl.BlockSpec((_BLK, D), lambda i: (i, 0))],
    out_specs=[pl.BlockSpec((_BLK, D), lambda i: (i, 0)),
               pl.BlockSpec((_BLK, 1), lambda i: (i, 0))],
    out_shape=[jax.ShapeDtypeStruct((NP, D), jnp.float32),
               jax.ShapeDtypeStruct((NP, 1), jnp.float32)],
)


def _mid_body(p0_ref, p1_ref, dinv_ref, b_ref, w_ref, gout_ref):
    h = jnp.maximum(
        dinv_ref[...] * (p0_ref[...] + p1_ref[...]) + b_ref[...], 0.0)
    gout_ref[...] = dinv_ref[...] * jnp.dot(
        h, w_ref[...], preferred_element_type=jnp.float32)


_mid = pl.pallas_call(
    _mid_body,
    grid=(_NBLK,),
    in_specs=[pl.BlockSpec((_BLK, D), lambda i: (i, 0)),
              pl.BlockSpec((_BLK, D), lambda i: (i, 0)),
              pl.BlockSpec((_BLK, 1), lambda i: (i, 0)),
              pl.BlockSpec((1, D), lambda i: (0, 0)),
              pl.BlockSpec((D, D), lambda i: (0, 0))],
    out_specs=pl.BlockSpec((_BLK, D), lambda i: (i, 0)),
    out_shape=jax.ShapeDtypeStruct((NP, D), jnp.float32),
)


def _fin_body(p0_ref, p1_ref, dinv_ref, b_ref, wab_ref, blin_ref,
              sa_ref, sb_ref):
    h = jnp.maximum(
        dinv_ref[...] * (p0_ref[...] + p1_ref[...]) + b_ref[...], 0.0)
    s = jnp.dot(h, wab_ref[...], preferred_element_type=jnp.float32)
    sa_ref[...] = s[:, 0:1] + blin_ref[0, 0]
    sb_ref[...] = s[:, 1:2]


_fin = pl.pallas_call(
    _fin_body,
    grid=(_NBLK,),
    in_specs=[pl.BlockSpec((_BLK, D), lambda i: (i, 0)),
              pl.BlockSpec((_BLK, D), lambda i: (i, 0)),
              pl.BlockSpec((_BLK, 1), lambda i: (i, 0)),
              pl.BlockSpec((1, D), lambda i: (0, 0)),
              pl.BlockSpec((D, 2), lambda i: (0, 0)),
              pl.BlockSpec((1, 1), lambda i: (0, 0))],
    out_specs=[pl.BlockSpec((_BLK, 1), lambda i: (i, 0)),
               pl.BlockSpec((_BLK, 1), lambda i: (i, 0))],
    out_shape=[jax.ShapeDtypeStruct((NP, 1), jnp.float32),
               jax.ShapeDtypeStruct((NP, 1), jnp.float32)],
)


def kernel(x, edge_index, W1, b1, W2, b2, Wlin, blin):
    src = edge_index[0]
    dst = edge_index[1]
    src4 = src.reshape(NW, NPH, CPP, K)
    dst4 = dst.reshape(NW, NPH, CPP, K)
    xp = jnp.pad(x, ((0, NP - N), (0, 0)))
    zeros_d = jnp.zeros((NP, D), jnp.float32)

    dp = _deg(dst)
    h1raw = _mm(xp, W1)
    degcol = _degsum(dp).reshape(NP, 1)
    g1, dinv = _s1(degcol, h1raw)

    p = _conv(g1, zeros_d, src4, dst4)
    g2 = _mid(p[0], p[1], dinv, b1.reshape(1, D), W2)

    p2 = _conv(g2, zeros_d, src4, dst4)
    wab = jnp.concatenate([Wlin[:D], Wlin[D:]], axis=1)
    sa, sb = _fin(p2[0], p2[1], dinv, b2.reshape(1, D), wab,
                  blin.reshape(1, 1))

    return _escore(sa.reshape(NP), sb.reshape(NP), src, dst)


# speedup vs baseline: 34.5563x; 1.0585x over previous
import dataclasses
import functools

import jax
import jax.numpy as jnp
from jax import lax
from jax.experimental import pallas as pl
from jax.experimental.pallas import tpu as pltpu
from jax.experimental.pallas import tpu_sc as plsc

N = 10000
NP = 10240
E = 320000
D = 128

NC = 2
NS = 16
NW = NC * NS
EPW = E // NW
K = 40
NCHUNK = EPW // K
NPH = 5
CPP = NCHUNK // NPH
NBUF = 5
RPW = NP // NS
DEGW = 16

_mesh = plsc.VectorSubcoreMesh(core_axis_name="c", subcore_axis_name="s",
                               num_cores=NC, num_subcores=NS)

_cp = pltpu.CompilerParams()
if "needs_layout_passes" in pltpu.CompilerParams.__dataclass_fields__:
    _cp = dataclasses.replace(_cp, needs_layout_passes=False)


def _wid():
    return lax.axis_index("s") * NC + lax.axis_index("c")


@functools.partial(
    pl.kernel,
    out_type=jax.ShapeDtypeStruct((NW, NP), jnp.float32),
    mesh=_mesh,
    scratch_types=[
        pltpu.VMEM((EPW,), jnp.int32),
        pltpu.VMEM((NP,), jnp.float32),
    ],
    compiler_params=_cp,
)
def _deg(dst_hbm, dp_hbm, idx_v, deg_v):
    wid = _wid()
    zero16 = jnp.zeros((16,), jnp.float32)
    one16 = jnp.ones((16,), jnp.float32)

    @pl.loop(0, NP, step=16)
    def _(i):
        deg_v[pl.ds(i, 16)] = zero16

    pltpu.sync_copy(dst_hbm.at[pl.ds(wid * EPW, EPW)], idx_v)

    @pl.loop(0, EPW, step=16)
    def _(j):
        iv = idx_v[pl.ds(j, 16)]
        plsc.addupdate_scatter(deg_v, [iv], one16)

    pltpu.sync_copy(deg_v, dp_hbm.at[wid])


@functools.partial(
    pl.kernel,
    out_type=jax.ShapeDtypeStruct((NC, NP, D), jnp.float32),
    mesh=_mesh,
    scratch_types=[
        pltpu.VMEM((CPP, K), jnp.int32),
        pltpu.VMEM((CPP, K), jnp.int32),
    ] + [pltpu.VMEM((K, D), jnp.float32)] * NBUF
      + [pltpu.SemaphoreType.DMA] * NBUF
      + [pltpu.VMEM_SHARED((NP, D), jnp.float32)],
)
def _conv(g_hbm, zeros_hbm, src4_hbm, dst4_hbm, p_hbm, sidx_v, didx_v,
          *rest):
    bufs = rest[:NBUF]
    sems = rest[NBUF:2 * NBUF]
    acc = rest[2 * NBUF]
    c = lax.axis_index("c")
    s = lax.axis_index("s")
    wid = _wid()

    @pl.when(c == 0)
    def _():
        pltpu.sync_copy(g_hbm.at[pl.ds(s * RPW, RPW)],
                        acc.at[pl.ds(s * RPW, RPW)])

    @pl.when(c == 1)
    def _():
        pltpu.sync_copy(zeros_hbm.at[pl.ds(s * RPW, RPW)],
                        acc.at[pl.ds(s * RPW, RPW)])

    plsc.subcore_barrier()

    @pl.loop(0, NPH)
    def _(t):
        pltpu.sync_copy(src4_hbm.at[wid, t], sidx_v)
        pltpu.sync_copy(dst4_hbm.at[wid, t], didx_v)
        for b in range(NBUF - 1):
            pltpu.async_copy(g_hbm.at[sidx_v.at[b]], bufs[b], sems[b])

        @pl.loop(0, CPP // NBUF)
        def _(g):
            j0 = g * NBUF
            for b in range(NBUF):
                j = j0 + b
                pltpu.make_async_copy(
                    g_hbm.at[sidx_v.at[j]], bufs[b], sems[b]).wait()
                nxt = j + NBUF - 1

                @pl.when(nxt < CPP)
                def _(b=b, nxt=nxt):
                    pltpu.async_copy(
                        g_hbm.at[sidx_v.at[nxt]],
                        bufs[(b - 1) % NBUF], sems[(b - 1) % NBUF])

                pltpu.sync_copy(bufs[b], acc.at[didx_v.at[j]], add=True)

    plsc.subcore_barrier()
    pltpu.sync_copy(acc.at[pl.ds(s * RPW, RPW)],
                    p_hbm.at[c, pl.ds(s * RPW, RPW)])


@functools.partial(
    pl.kernel,
    out_type=jax.ShapeDtypeStruct((E,), jnp.float32),
    mesh=_mesh,
    scratch_types=[
        pltpu.VMEM((NP,), jnp.float32),
        pltpu.VMEM((NP,), jnp.float32),
        pltpu.VMEM((EPW,), jnp.int32),
        pltpu.VMEM((EPW,), jnp.int32),
        pltpu.VMEM((EPW,), jnp.float32),
    ],
    compiler_params=_cp,
)
def _escore(sa_hbm, sb_hbm, src_hbm, dst_hbm, out_hbm, sa_v, sb_v, src_v,
            dst_v, out_v):
    wid = _wid()
    base = wid * EPW
    pltpu.sync_copy(sa_hbm, sa_v)
    pltpu.sync_copy(sb_hbm, sb_v)
    pltpu.sync_copy(src_hbm.at[pl.ds(base, EPW)], src_v)
    pltpu.sync_copy(dst_hbm.at[pl.ds(base, EPW)], dst_v)

    @pl.loop(0, EPW, step=16)
    def _(j):
        iv_s = src_v[pl.ds(j, 16)]
        iv_d = dst_v[pl.ds(j, 16)]
        va = plsc.load_gather(sa_v, [iv_s])
        vb = plsc.load_gather(sb_v, [iv_d])
        out_v[pl.ds(j, 16)] = va + vb

    pltpu.sync_copy(out_v, out_hbm.at[pl.ds(base, EPW)])


_BLK = 1024
_NBLK = NP // _BLK


def _tc1_body(x_ref, w_ref, dp_ref, g_ref, dinv_ref):
    h = jnp.dot(x_ref[...], w_ref[...], preferred_element_type=jnp.float32)
    degrow = jnp.sum(dp_ref[...], axis=0, keepdims=True) + 1.0
    dinvrow = lax.rsqrt(jnp.maximum(degrow, 1e-12))
    dinv = jnp.swapaxes(dinvrow, 0, 1)
    dinv_ref[...] = dinv
    g_ref[...] = h * dinv


_tc1 = pl.pallas_call(
    _tc1_body,
    grid=(_NBLK,),
    in_specs=[pl.BlockSpec((_BLK, D), lambda i: (i, 0)),
              pl.BlockSpec((D, D), lambda i: (0, 0)),
              pl.BlockSpec((NW, _BLK), lambda i: (0, i))],
    out_specs=[pl.BlockSpec((_BLK, D), lambda i: (i, 0)),
               pl.BlockSpec((_BLK, 1), lambda i: (i, 0))],
    out_shape=[jax.ShapeDtypeStruct((NP, D), jnp.float32),
               jax.ShapeDtypeStruct((NP, 1), jnp.float32)],
)


def _mid_body(p0_ref, p1_ref, dinv_ref, b_ref, w_ref, gout_ref):
    h = jnp.maximum(
        dinv_ref[...] * (p0_ref[...] + p1_ref[...]) + b_ref[...], 0.0)
    gout_ref[...] = dinv_ref[...] * jnp.dot(
        h, w_ref[...], preferred_element_type=jnp.float32)


_mid = pl.pallas_call(
    _mid_body,
    grid=(_NBLK,),
    in_specs=[pl.BlockSpec((_BLK, D), lambda i: (i, 0)),
              pl.BlockSpec((_BLK, D), lambda i: (i, 0)),
              pl.BlockSpec((_BLK, 1), lambda i: (i, 0)),
              pl.BlockSpec((1, D), lambda i: (0, 0)),
              pl.BlockSpec((D, D), lambda i: (0, 0))],
    out_specs=pl.BlockSpec((_BLK, D), lambda i: (i, 0)),
    out_shape=jax.ShapeDtypeStruct((NP, D), jnp.float32),
)


def _fin_body(p0_ref, p1_ref, dinv_ref, b_ref, wab_ref, blin_ref,
              sa_ref, sb_ref):
    h = jnp.maximum(
        dinv_ref[...] * (p0_ref[...] + p1_ref[...]) + b_ref[...], 0.0)
    s = jnp.dot(h, wab_ref[...], preferred_element_type=jnp.float32)
    sa_ref[...] = s[:, 0:1] + blin_ref[0, 0]
    sb_ref[...] = s[:, 1:2]


_fin = pl.pallas_call(
    _fin_body,
    grid=(_NBLK,),
    in_specs=[pl.BlockSpec((_BLK, D), lambda i: (i, 0)),
              pl.BlockSpec((_BLK, D), lambda i: (i, 0)),
              pl.BlockSpec((_BLK, 1), lambda i: (i, 0)),
              pl.BlockSpec((1, D), lambda i: (0, 0)),
              pl.BlockSpec((D, 2), lambda i: (0, 0)),
              pl.BlockSpec((1, 1), lambda i: (0, 0))],
    out_specs=[pl.BlockSpec((_BLK, 1), lambda i: (i, 0)),
               pl.BlockSpec((_BLK, 1), lambda i: (i, 0))],
    out_shape=[jax.ShapeDtypeStruct((NP, 1), jnp.float32),
               jax.ShapeDtypeStruct((NP, 1), jnp.float32)],
)


def kernel(x, edge_index, W1, b1, W2, b2, Wlin, blin):
    src = edge_index[0]
    dst = edge_index[1]
    src4 = src.reshape(NW, NPH, CPP, K)
    dst4 = dst.reshape(NW, NPH, CPP, K)
    xp = jnp.pad(x, ((0, NP - N), (0, 0)))
    zeros_d = jnp.zeros((NP, D), jnp.float32)

    dp = _deg(dst)
    g1, dinv = _tc1(xp, W1, dp)

    p = _conv(g1, zeros_d, src4, dst4)
    g2 = _mid(p[0], p[1], dinv, b1.reshape(1, D), W2)

    p2 = _conv(g2, zeros_d, src4, dst4)
    wab = jnp.concatenate([Wlin[:D], Wlin[D:]], axis=1)
    sa, sb = _fin(p2[0], p2[1], dinv, b2.reshape(1, D), wab,
                  blin.reshape(1, 1))

    return _escore(sa.reshape(NP), sb.reshape(NP), src, dst)
